# baseline (speedup 1.0000x reference)
import functools

import jax
import jax.numpy as jnp
from jax import lax
from jax.experimental import pallas as pl
from jax.experimental.pallas import tpu as pltpu
from jax.experimental.pallas import tpu_sc as plsc


def _stage1_body(f_ref, w1_ref, out_ref):
    c = f_ref.shape[1]
    w1 = w1_ref[...]
    wb = (w1[:, c:2 * c] + w1[:, 2 * c:]).astype(jnp.bfloat16)
    r = lax.dot_general(f_ref[...].astype(jnp.bfloat16), wb,
                        (((1,), (1,)), ((), ())),
                        preferred_element_type=jnp.float32)
    hh = r.shape[1] // 2
    lo = lax.bitcast_convert_type(
        r[:, :hh].astype(jnp.bfloat16).astype(jnp.float32), jnp.uint32)
    hi = lax.bitcast_convert_type(
        r[:, hh:].astype(jnp.bfloat16).astype(jnp.float32), jnp.uint32)
    word = hi | lax.shift_right_logical(lo, jnp.uint32(16))
    out_ref[...] = lax.bitcast_convert_type(word, jnp.float32)


@functools.cache
def _make_gather(nk, hw, chunk):
    info = plsc.get_sparse_core_info()
    nw = info.num_cores * info.num_subcores
    per_w = nk // nw
    nch = per_w // chunk
    mesh = plsc.VectorSubcoreMesh(core_axis_name="c", subcore_axis_name="s")
    nbuf = 4

    def body(table_hbm, idx_hbm, out_hbm, idx_v, rows_v, *sems):
        gsems, osems = sems[:nbuf], sems[nbuf:]
        wid = lax.axis_index("s") * info.num_cores + lax.axis_index("c")
        base = wid * per_w
        pltpu.sync_copy(idx_hbm.at[pl.ds(base, per_w)], idx_v)

        def start_gather(c):
            pltpu.async_copy(
                table_hbm.at[idx_v.at[pl.ds(c * chunk, chunk)]],
                rows_v.at[c % nbuf], gsems[c % nbuf])

        for c in range(min(2, nch)):
            start_gather(c)
        for c in range(nch):
            p = c % nbuf
            pltpu.make_async_copy(
                table_hbm.at[idx_v.at[pl.ds(c * chunk, chunk)]],
                rows_v.at[p], gsems[p]).wait()
            pltpu.async_copy(
                rows_v.at[p], out_hbm.at[pl.ds(base + c * chunk, chunk)],
                osems[p])
            nxt = c + 2
            if nxt < nch:
                q = nxt % nbuf
                if nxt >= nbuf:
                    pltpu.make_async_copy(
                        rows_v.at[q],
                        out_hbm.at[pl.ds(base + (nxt - nbuf) * chunk, chunk)],
                        osems[q]).wait()
                start_gather(nxt)
        for c in range(max(nch - nbuf, 0), nch):
            p = c % nbuf
            pltpu.make_async_copy(
                rows_v.at[p], out_hbm.at[pl.ds(base + c * chunk, chunk)],
                osems[p]).wait()

    return pl.kernel(
        body,
        out_type=jax.ShapeDtypeStruct((nk, hw), jnp.float32),
        mesh=mesh,
        scratch_types=[
            pltpu.VMEM((per_w,), jnp.int32),
            pltpu.VMEM((nbuf, chunk, hw), jnp.float32),
        ] + [pltpu.SemaphoreType.DMA] * (2 * nbuf),
    )


def _stage3_body(f_ref, g_ref, w1_ref, b1_ref, w2_ref, b2_ref, w3_ref,
                 b3_ref, out_ref):
    bn, c = f_ref.shape
    k = g_ref.shape[0] // bn
    h = w2_ref.shape[0]
    hh = h // 2
    w1 = w1_ref[...]
    wa = (w1[:, :c] - w1[:, 2 * c:]).astype(jnp.bfloat16)
    a = lax.dot_general(f_ref[...].astype(jnp.bfloat16), wa,
                        (((1,), (1,)), ((), ())),
                        preferred_element_type=jnp.float32) + b1_ref[...]
    word = lax.bitcast_convert_type(g_ref[...], jnp.uint32)
    g_lo = lax.bitcast_convert_type(
        lax.shift_left(word, jnp.uint32(16)), jnp.float32)
    g_hi = lax.bitcast_convert_type(word & jnp.uint32(0xFFFF0000), jnp.float32)
    g_all = jnp.concatenate([g_lo, g_hi], axis=1).reshape(bn, k, h)
    h1 = jnp.maximum(g_all + a[:, None, :], 0.0).astype(jnp.bfloat16)
    w2 = w2_ref[...]
    pooled = None
    for j in range(k):
        h2j = lax.dot_general(h1[:, j, :], w2, (((1,), (1,)), ((), ())),
                              preferred_element_type=jnp.float32)
        pooled = h2j if pooled is None else jnp.maximum(pooled, h2j)
    pooled = jnp.maximum(pooled + b2_ref[...], 0.0)
    out_ref[...] = lax.dot_general(
        pooled, w3_ref[...], (((1,), (1,)), ((), ())),
        preferred_element_type=jnp.float32) + b3_ref[...]


def kernel(f, knn_idx, W1, b1, W2, b2, W3, b3):
    B, N, C = f.shape
    K = knn_idx.shape[-1]
    H = W1.shape[0]
    O = W3.shape[0]
    NK = N * K
    BN = 400
    S = 5
    NSL = N // S

    f2 = f.reshape(N, C)
    idx = knn_idx.reshape(NK).astype(jnp.int32)

    table = pl.pallas_call(
        _stage1_body,
        out_shape=jax.ShapeDtypeStruct((N, H // 2), jnp.float32),
    )(f2, W1)

    gather = _make_gather(NK // S, H // 2, 200)
    w2b = W2.astype(jnp.bfloat16)
    b1r, b2r, b3r = b1.reshape(1, H), b2.reshape(1, H), b3.reshape(1, O)

    stage3 = pl.pallas_call(
        _stage3_body,
        grid=(NSL // BN,),
        in_specs=[
            pl.BlockSpec((BN, C), lambda i: (i, 0)),
            pl.BlockSpec((BN * K, H // 2), lambda i: (i, 0)),
            pl.BlockSpec((H, 3 * C), lambda i: (0, 0)),
            pl.BlockSpec((1, H), lambda i: (0, 0)),
            pl.BlockSpec((H, H), lambda i: (0, 0)),
            pl.BlockSpec((1, H), lambda i: (0, 0)),
            pl.BlockSpec((O, H), lambda i: (0, 0)),
            pl.BlockSpec((1, O), lambda i: (0, 0)),
        ],
        out_specs=pl.BlockSpec((BN, O), lambda i: (i, 0)),
        out_shape=jax.ShapeDtypeStruct((NSL, O), jnp.float32),
    )

    gs = [gather(table,
                 lax.slice(idx, (s * NK // S,), ((s + 1) * NK // S,)))
          for s in range(S)]
    outs = [stage3(lax.slice(f2, (s * NSL, 0), ((s + 1) * NSL, C)), gs[s],
                   W1, b1r, w2b, b2r, W3, b3r)
            for s in range(S)]
    out = jnp.concatenate(outs, axis=0)

    return out.reshape(B, N, O)


# speedup vs baseline: 1.3236x; 1.3236x over previous
import functools

import jax
import jax.numpy as jnp
from jax import lax
from jax.experimental import pallas as pl
from jax.experimental.pallas import tpu as pltpu
from jax.experimental.pallas import tpu_sc as plsc


def _stage1_body(f_ref, w1_ref, out_ref):
    c = f_ref.shape[1]
    w1 = w1_ref[...]
    wb = (w1[:, c:2 * c] + w1[:, 2 * c:]).astype(jnp.bfloat16)
    r = lax.dot_general(f_ref[...].astype(jnp.bfloat16), wb,
                        (((1,), (1,)), ((), ())),
                        preferred_element_type=jnp.float32)
    hh = r.shape[1] // 2
    lo = lax.bitcast_convert_type(
        r[:, :hh].astype(jnp.bfloat16).astype(jnp.float32), jnp.uint32)
    hi = lax.bitcast_convert_type(
        r[:, hh:].astype(jnp.bfloat16).astype(jnp.float32), jnp.uint32)
    word = hi | lax.shift_right_logical(lo, jnp.uint32(16))
    out_ref[...] = lax.bitcast_convert_type(word, jnp.float32)


@functools.cache
def _make_gather(nk, hw, chunk):
    info = plsc.get_sparse_core_info()
    nw = info.num_cores * info.num_subcores
    per_w = nk // nw
    nch = per_w // chunk
    mesh = plsc.VectorSubcoreMesh(core_axis_name="c", subcore_axis_name="s")
    nbuf = 4

    def body(table_hbm, idx_hbm, out_hbm, idx_v, rows_v, *sems):
        gsems, osems = sems[:nbuf], sems[nbuf:]
        wid = lax.axis_index("s") * info.num_cores + lax.axis_index("c")
        base = wid * per_w
        pltpu.sync_copy(idx_hbm.at[pl.ds(base, per_w)], idx_v)

        def start_gather(c):
            pltpu.async_copy(
                table_hbm.at[idx_v.at[pl.ds(c * chunk, chunk)]],
                rows_v.at[c % nbuf], gsems[c % nbuf])

        for c in range(min(2, nch)):
            start_gather(c)
        for c in range(nch):
            p = c % nbuf
            pltpu.make_async_copy(
                table_hbm.at[idx_v.at[pl.ds(c * chunk, chunk)]],
                rows_v.at[p], gsems[p]).wait()
            pltpu.async_copy(
                rows_v.at[p], out_hbm.at[pl.ds(base + c * chunk, chunk)],
                osems[p])
            nxt = c + 2
            if nxt < nch:
                q = nxt % nbuf
                if nxt >= nbuf:
                    pltpu.make_async_copy(
                        rows_v.at[q],
                        out_hbm.at[pl.ds(base + (nxt - nbuf) * chunk, chunk)],
                        osems[q]).wait()
                start_gather(nxt)
        for c in range(max(nch - nbuf, 0), nch):
            p = c % nbuf
            pltpu.make_async_copy(
                rows_v.at[p], out_hbm.at[pl.ds(base + c * chunk, chunk)],
                osems[p]).wait()

    return pl.kernel(
        body,
        out_type=jax.ShapeDtypeStruct((nk, hw), jnp.float32),
        mesh=mesh,
        scratch_types=[
            pltpu.VMEM((per_w,), jnp.int32),
            pltpu.VMEM((nbuf, chunk, hw), jnp.float32),
        ] + [pltpu.SemaphoreType.DMA] * (2 * nbuf),
    )


def _stage3_body(f_ref, g_ref, w1_ref, b1_ref, w2_ref, b2_ref, w3_ref,
                 b3_ref, out_ref):
    bn, c = f_ref.shape
    k = g_ref.shape[0] // bn
    h = w2_ref.shape[0]
    hh = h // 2
    w1 = w1_ref[...]
    wa = (w1[:, :c] - w1[:, 2 * c:]).astype(jnp.bfloat16)
    a = lax.dot_general(f_ref[...].astype(jnp.bfloat16), wa,
                        (((1,), (1,)), ((), ())),
                        preferred_element_type=jnp.float32) + b1_ref[...]
    word = lax.bitcast_convert_type(g_ref[...], jnp.uint32)
    g_lo = lax.bitcast_convert_type(
        lax.shift_left(word, jnp.uint32(16)), jnp.float32)
    g_hi = lax.bitcast_convert_type(word & jnp.uint32(0xFFFF0000), jnp.float32)
    g_all = jnp.concatenate([g_lo, g_hi], axis=1).reshape(bn, k, h)
    h1 = jnp.maximum(g_all + a[:, None, :], 0.0)
    h2 = lax.dot_general(h1.reshape(bn * k, h).astype(jnp.bfloat16),
                         w2_ref[...], (((1,), (1,)), ((), ())),
                         preferred_element_type=jnp.float32) + b2_ref[...]
    h2 = jnp.maximum(h2, 0.0)
    pooled = jnp.max(h2.reshape(bn, k, h), axis=1)
    out_ref[...] = lax.dot_general(
        pooled, w3_ref[...], (((1,), (1,)), ((), ())),
        preferred_element_type=jnp.float32) + b3_ref[...]


def kernel(f, knn_idx, W1, b1, W2, b2, W3, b3):
    B, N, C = f.shape
    K = knn_idx.shape[-1]
    H = W1.shape[0]
    O = W3.shape[0]
    NK = N * K
    BN = 400
    S = 5
    NSL = N // S

    f2 = f.reshape(N, C)
    idx = knn_idx.reshape(NK).astype(jnp.int32)

    table = pl.pallas_call(
        _stage1_body,
        out_shape=jax.ShapeDtypeStruct((N, H // 2), jnp.float32),
    )(f2, W1)

    gather = _make_gather(NK // S, H // 2, 200)
    w2b = W2.astype(jnp.bfloat16)
    b1r, b2r, b3r = b1.reshape(1, H), b2.reshape(1, H), b3.reshape(1, O)

    stage3 = pl.pallas_call(
        _stage3_body,
        grid=(NSL // BN,),
        in_specs=[
            pl.BlockSpec((BN, C), lambda i: (i, 0)),
            pl.BlockSpec((BN * K, H // 2), lambda i: (i, 0)),
            pl.BlockSpec((H, 3 * C), lambda i: (0, 0)),
            pl.BlockSpec((1, H), lambda i: (0, 0)),
            pl.BlockSpec((H, H), lambda i: (0, 0)),
            pl.BlockSpec((1, H), lambda i: (0, 0)),
            pl.BlockSpec((O, H), lambda i: (0, 0)),
            pl.BlockSpec((1, O), lambda i: (0, 0)),
        ],
        out_specs=pl.BlockSpec((BN, O), lambda i: (i, 0)),
        out_shape=jax.ShapeDtypeStruct((NSL, O), jnp.float32),
    )

    gs = [gather(table,
                 lax.slice(idx, (s * NK // S,), ((s + 1) * NK // S,)))
          for s in range(S)]
    outs = [stage3(lax.slice(f2, (s * NSL, 0), ((s + 1) * NSL, C)), gs[s],
                   W1, b1r, w2b, b2r, W3, b3r)
            for s in range(S)]
    out = jnp.concatenate(outs, axis=0)

    return out.reshape(B, N, O)


# speedup vs baseline: 1.3556x; 1.0241x over previous
import functools

import jax
import jax.numpy as jnp
from jax import lax
from jax.experimental import pallas as pl
from jax.experimental.pallas import tpu as pltpu
from jax.experimental.pallas import tpu_sc as plsc


def _stage1_body(f_ref, w1_ref, out_ref):
    c = f_ref.shape[1]
    w1 = w1_ref[...]
    wb = (w1[:, c:2 * c] + w1[:, 2 * c:]).astype(jnp.bfloat16)
    r = lax.dot_general(f_ref[...].astype(jnp.bfloat16), wb,
                        (((1,), (1,)), ((), ())),
                        preferred_element_type=jnp.float32)
    hh = r.shape[1] // 2
    lo = lax.bitcast_convert_type(
        r[:, :hh].astype(jnp.bfloat16).astype(jnp.float32), jnp.uint32)
    hi = lax.bitcast_convert_type(
        r[:, hh:].astype(jnp.bfloat16).astype(jnp.float32), jnp.uint32)
    word = hi | lax.shift_right_logical(lo, jnp.uint32(16))
    out_ref[...] = lax.bitcast_convert_type(word, jnp.float32)


@functools.cache
def _make_gather(nk, hw, chunk):
    info = plsc.get_sparse_core_info()
    nw = info.num_cores * info.num_subcores
    per_w = nk // nw
    nch = per_w // chunk
    mesh = plsc.VectorSubcoreMesh(core_axis_name="c", subcore_axis_name="s")
    nbuf = 4

    def body(table_hbm, idx_hbm, out_hbm, idx_v, rows_v, *sems):
        gsems, osems = sems[:nbuf], sems[nbuf:]
        wid = lax.axis_index("s") * info.num_cores + lax.axis_index("c")
        base = wid * per_w
        pltpu.sync_copy(idx_hbm.at[pl.ds(base, per_w)], idx_v)

        def start_gather(c):
            pltpu.async_copy(
                table_hbm.at[idx_v.at[pl.ds(c * chunk, chunk)]],
                rows_v.at[c % nbuf], gsems[c % nbuf])

        for c in range(min(2, nch)):
            start_gather(c)
        for c in range(nch):
            p = c % nbuf
            pltpu.make_async_copy(
                table_hbm.at[idx_v.at[pl.ds(c * chunk, chunk)]],
                rows_v.at[p], gsems[p]).wait()
            pltpu.async_copy(
                rows_v.at[p], out_hbm.at[pl.ds(base + c * chunk, chunk)],
                osems[p])
            nxt = c + 2
            if nxt < nch:
                q = nxt % nbuf
                if nxt >= nbuf:
                    pltpu.make_async_copy(
                        rows_v.at[q],
                        out_hbm.at[pl.ds(base + (nxt - nbuf) * chunk, chunk)],
                        osems[q]).wait()
                start_gather(nxt)
        for c in range(max(nch - nbuf, 0), nch):
            p = c % nbuf
            pltpu.make_async_copy(
                rows_v.at[p], out_hbm.at[pl.ds(base + c * chunk, chunk)],
                osems[p]).wait()

    return pl.kernel(
        body,
        out_type=jax.ShapeDtypeStruct((nk, hw), jnp.float32),
        mesh=mesh,
        scratch_types=[
            pltpu.VMEM((per_w,), jnp.int32),
            pltpu.VMEM((nbuf, chunk, hw), jnp.float32),
        ] + [pltpu.SemaphoreType.DMA] * (2 * nbuf),
    )


def _stage3_body(f_ref, g_ref, w1_ref, b1_ref, w2_ref, b2_ref, w3_ref,
                 b3_ref, out_ref):
    bn, c = f_ref.shape
    k = g_ref.shape[0] // bn
    h = w2_ref.shape[0]
    hh = h // 2
    w1 = w1_ref[...]
    wa = (w1[:, :c] - w1[:, 2 * c:]).astype(jnp.bfloat16)
    a = lax.dot_general(f_ref[...].astype(jnp.bfloat16), wa,
                        (((1,), (1,)), ((), ())),
                        preferred_element_type=jnp.float32) + b1_ref[...]
    word = lax.bitcast_convert_type(g_ref[...], jnp.uint32)
    g_lo = lax.bitcast_convert_type(
        lax.shift_left(word, jnp.uint32(16)), jnp.float32)
    g_hi = lax.bitcast_convert_type(word & jnp.uint32(0xFFFF0000), jnp.float32)
    h1_lo = jnp.maximum(g_lo.reshape(bn, k, hh) + a[:, None, :hh],
                        0.0).reshape(bn * k, hh).astype(jnp.bfloat16)
    h1_hi = jnp.maximum(g_hi.reshape(bn, k, hh) + a[:, None, hh:],
                        0.0).reshape(bn * k, hh).astype(jnp.bfloat16)
    w2 = w2_ref[...]
    h2 = (lax.dot_general(h1_lo, w2[:, :hh], (((1,), (1,)), ((), ())),
                          preferred_element_type=jnp.float32)
          + lax.dot_general(h1_hi, w2[:, hh:], (((1,), (1,)), ((), ())),
                            preferred_element_type=jnp.float32)
          + b2_ref[...])
    h2 = jnp.maximum(h2, 0.0)
    pooled = jnp.max(h2.reshape(bn, k, h), axis=1)
    out_ref[...] = lax.dot_general(
        pooled, w3_ref[...], (((1,), (1,)), ((), ())),
        preferred_element_type=jnp.float32) + b3_ref[...]


def kernel(f, knn_idx, W1, b1, W2, b2, W3, b3):
    B, N, C = f.shape
    K = knn_idx.shape[-1]
    H = W1.shape[0]
    O = W3.shape[0]
    NK = N * K
    BN = 1000
    S = 5
    NSL = N // S

    f2 = f.reshape(N, C)
    idx = knn_idx.reshape(NK).astype(jnp.int32)

    table = pl.pallas_call(
        _stage1_body,
        out_shape=jax.ShapeDtypeStruct((N, H // 2), jnp.float32),
    )(f2, W1)

    gather = _make_gather(NK // S, H // 2, 200)
    w2b = W2.astype(jnp.bfloat16)
    b1r, b2r, b3r = b1.reshape(1, H), b2.reshape(1, H), b3.reshape(1, O)

    stage3 = pl.pallas_call(
        _stage3_body,
        grid=(NSL // BN,),
        in_specs=[
            pl.BlockSpec((BN, C), lambda i: (i, 0)),
            pl.BlockSpec((BN * K, H // 2), lambda i: (i, 0)),
            pl.BlockSpec((H, 3 * C), lambda i: (0, 0)),
            pl.BlockSpec((1, H), lambda i: (0, 0)),
            pl.BlockSpec((H, H), lambda i: (0, 0)),
            pl.BlockSpec((1, H), lambda i: (0, 0)),
            pl.BlockSpec((O, H), lambda i: (0, 0)),
            pl.BlockSpec((1, O), lambda i: (0, 0)),
        ],
        out_specs=pl.BlockSpec((BN, O), lambda i: (i, 0)),
        out_shape=jax.ShapeDtypeStruct((NSL, O), jnp.float32),
    )

    gs = [gather(table,
                 lax.slice(idx, (s * NK // S,), ((s + 1) * NK // S,)))
          for s in range(S)]
    outs = [stage3(lax.slice(f2, (s * NSL, 0), ((s + 1) * NSL, C)), gs[s],
                   W1, b1r, w2b, b2r, W3, b3r)
            for s in range(S)]
    out = jnp.concatenate(outs, axis=0)

    return out.reshape(B, N, O)


# speedup vs baseline: 1.4497x; 1.0694x over previous
import functools

import jax
import jax.numpy as jnp
from jax import lax
from jax.experimental import pallas as pl
from jax.experimental.pallas import tpu as pltpu
from jax.experimental.pallas import tpu_sc as plsc


def _stage1_body(f_ref, w1_ref, out_ref):
    c = f_ref.shape[1]
    w1 = w1_ref[...]
    wb = (w1[:, c:2 * c] + w1[:, 2 * c:]).astype(jnp.bfloat16)
    r = lax.dot_general(f_ref[...].astype(jnp.bfloat16), wb,
                        (((1,), (1,)), ((), ())),
                        preferred_element_type=jnp.float32)
    hh = r.shape[1] // 2
    lo = lax.bitcast_convert_type(
        r[:, :hh].astype(jnp.bfloat16).astype(jnp.float32), jnp.uint32)
    hi = lax.bitcast_convert_type(
        r[:, hh:].astype(jnp.bfloat16).astype(jnp.float32), jnp.uint32)
    word = hi | lax.shift_right_logical(lo, jnp.uint32(16))
    out_ref[...] = lax.bitcast_convert_type(word, jnp.float32)


@functools.cache
def _make_gather(nk, hw, chunk):
    info = plsc.get_sparse_core_info()
    nw = info.num_cores * info.num_subcores
    per_w = nk // nw
    nch = per_w // chunk
    mesh = plsc.VectorSubcoreMesh(core_axis_name="c", subcore_axis_name="s")
    nbuf = 4

    def body(table_hbm, idx_hbm, out_hbm, idx_v, rows_v, *sems):
        gsems, osems = sems[:nbuf], sems[nbuf:]
        wid = lax.axis_index("s") * info.num_cores + lax.axis_index("c")
        base = wid * per_w
        pltpu.sync_copy(idx_hbm.at[pl.ds(base, per_w)], idx_v)

        def start_gather(c):
            pltpu.async_copy(
                table_hbm.at[idx_v.at[pl.ds(c * chunk, chunk)]],
                rows_v.at[c % nbuf], gsems[c % nbuf])

        for c in range(min(2, nch)):
            start_gather(c)
        for c in range(nch):
            p = c % nbuf
            pltpu.make_async_copy(
                table_hbm.at[idx_v.at[pl.ds(c * chunk, chunk)]],
                rows_v.at[p], gsems[p]).wait()
            pltpu.async_copy(
                rows_v.at[p], out_hbm.at[pl.ds(base + c * chunk, chunk)],
                osems[p])
            nxt = c + 2
            if nxt < nch:
                q = nxt % nbuf
                if nxt >= nbuf:
                    pltpu.make_async_copy(
                        rows_v.at[q],
                        out_hbm.at[pl.ds(base + (nxt - nbuf) * chunk, chunk)],
                        osems[q]).wait()
                start_gather(nxt)
        for c in range(max(nch - nbuf, 0), nch):
            p = c % nbuf
            pltpu.make_async_copy(
                rows_v.at[p], out_hbm.at[pl.ds(base + c * chunk, chunk)],
                osems[p]).wait()

    return pl.kernel(
        body,
        out_type=jax.ShapeDtypeStruct((nk, hw), jnp.float32),
        mesh=mesh,
        scratch_types=[
            pltpu.VMEM((per_w,), jnp.int32),
            pltpu.VMEM((nbuf, chunk, hw), jnp.float32),
        ] + [pltpu.SemaphoreType.DMA] * (2 * nbuf),
    )


def _stage3_body(f_ref, g_ref, w1_ref, b1_ref, w2_ref, b2_ref, w3_ref,
                 b3_ref, out_ref):
    bn, c = f_ref.shape
    k = g_ref.shape[0]
    h = w2_ref.shape[0]
    hh = h // 2
    w1 = w1_ref[...]
    wa = (w1[:, :c] - w1[:, 2 * c:]).astype(jnp.bfloat16)
    a = lax.dot_general(f_ref[...].astype(jnp.bfloat16), wa,
                        (((1,), (1,)), ((), ())),
                        preferred_element_type=jnp.float32) + b1_ref[...]
    word = lax.bitcast_convert_type(g_ref[...], jnp.uint32)
    g_lo = lax.bitcast_convert_type(
        lax.shift_left(word, jnp.uint32(16)), jnp.float32)
    g_hi = lax.bitcast_convert_type(word & jnp.uint32(0xFFFF0000), jnp.float32)
    h1_lo = jnp.maximum(g_lo + a[None, :, :hh],
                        0.0).reshape(k * bn, hh).astype(jnp.bfloat16)
    h1_hi = jnp.maximum(g_hi + a[None, :, hh:],
                        0.0).reshape(k * bn, hh).astype(jnp.bfloat16)
    w2 = w2_ref[...]
    h2 = (lax.dot_general(h1_lo, w2[:, :hh], (((1,), (1,)), ((), ())),
                          preferred_element_type=jnp.float32)
          + lax.dot_general(h1_hi, w2[:, hh:], (((1,), (1,)), ((), ())),
                            preferred_element_type=jnp.float32))
    pooled = jnp.max(h2.reshape(k, bn, h), axis=0)
    pooled = jnp.maximum(pooled + b2_ref[...], 0.0)
    out_ref[...] = lax.dot_general(
        pooled, w3_ref[...], (((1,), (1,)), ((), ())),
        preferred_element_type=jnp.float32) + b3_ref[...]


def kernel(f, knn_idx, W1, b1, W2, b2, W3, b3):
    B, N, C = f.shape
    K = knn_idx.shape[-1]
    H = W1.shape[0]
    O = W3.shape[0]
    NK = N * K
    BN = 1000
    S = 5
    NSL = N // S

    f2 = f.reshape(N, C)
    idx2 = knn_idx.reshape(N, K).astype(jnp.int32)

    table = pl.pallas_call(
        _stage1_body,
        out_shape=jax.ShapeDtypeStruct((N, H // 2), jnp.float32),
    )(f2, W1)

    gather = _make_gather(NK // S, H // 2, 200)
    w2b = W2.astype(jnp.bfloat16)
    b1r, b2r, b3r = b1.reshape(1, H), b2.reshape(1, H), b3.reshape(1, O)

    stage3 = pl.pallas_call(
        _stage3_body,
        grid=(NSL // BN,),
        in_specs=[
            pl.BlockSpec((BN, C), lambda i: (i, 0)),
            pl.BlockSpec((K, BN, H // 2), lambda i: (0, i, 0)),
            pl.BlockSpec((H, 3 * C), lambda i: (0, 0)),
            pl.BlockSpec((1, H), lambda i: (0, 0)),
            pl.BlockSpec((H, H), lambda i: (0, 0)),
            pl.BlockSpec((1, H), lambda i: (0, 0)),
            pl.BlockSpec((O, H), lambda i: (0, 0)),
            pl.BlockSpec((1, O), lambda i: (0, 0)),
        ],
        out_specs=pl.BlockSpec((BN, O), lambda i: (i, 0)),
        out_shape=jax.ShapeDtypeStruct((NSL, O), jnp.float32),
    )

    outs = []
    for s in range(S):
        idx_s = idx2[s * NSL:(s + 1) * NSL].T.reshape(NSL * K)
        g_s = gather(table, idx_s).reshape(K, NSL, H // 2)
        outs.append(stage3(
            lax.slice(f2, (s * NSL, 0), ((s + 1) * NSL, C)), g_s,
            W1, b1r, w2b, b2r, W3, b3r))
    out = jnp.concatenate(outs, axis=0)

    return out.reshape(B, N, O)
